# Initial kernel scaffold; baseline (speedup 1.0000x reference)
import jax
import jax.numpy as jnp
from jax.experimental import pallas as pl
from jax.experimental.pallas import tpu as pltpu

S = 2048
H = 1024
E = 8
K = 2
CAP = 768
THRESH = 0.5
SB = 256


def _router_body(h_ref, rw1_ref, rb1_ref, rw2_ref, rb2_ref,
                 iw1_ref, ib1_ref, iw2_ref, ib2_ref,
                 disp_ref, comb_ref, probs_ref, imp_ref, aux_ref,
                 cnt_ref, psum_ref, usum_ref):
    i = pl.program_id(0)
    nsteps = pl.num_programs(0)

    @pl.when(i == 0)
    def _init():
        cnt_ref[...] = jnp.zeros_like(cnt_ref)
        psum_ref[...] = jnp.zeros_like(psum_ref)
        usum_ref[...] = jnp.zeros_like(usum_ref)

    h = h_ref[...]

    rh = jnp.maximum(jnp.dot(h, rw1_ref[...],
                             preferred_element_type=jnp.float32)
                     + rb1_ref[...], 0.0)
    logits = jnp.dot(rh, rw2_ref[...],
                     preferred_element_type=jnp.float32) + rb2_ref[...]

    m = jnp.max(logits, axis=1, keepdims=True)
    ex = jnp.exp(logits - m)
    probs = ex / jnp.sum(ex, axis=1, keepdims=True)
    probs_ref[...] = probs

    eidx = jax.lax.broadcasted_iota(jnp.int32, (SB, E), 1)
    m1 = jnp.max(probs, axis=1, keepdims=True)
    i1 = jnp.min(jnp.where(probs == m1, eidx, E), axis=1, keepdims=True)
    sel1 = eidx == i1
    rest = jnp.where(sel1, -1.0, probs)
    m2 = jnp.max(rest, axis=1, keepdims=True)
    i2 = jnp.min(jnp.where(rest == m2, eidx, E), axis=1, keepdims=True)
    sel2 = eidx == i2
    sel = sel1 | sel2
    sel_f = sel.astype(jnp.float32)

    denom = m1 + m2 + 1e-8
    pnorm = jnp.where(sel1, m1 / denom, 0.0) + jnp.where(sel2, m2 / denom, 0.0)

    ih = jnp.maximum(jnp.dot(h, iw1_ref[...],
                             preferred_element_type=jnp.float32)
                     + ib1_ref[...], 0.0)
    il = jnp.dot(ih, iw2_ref[...],
                 preferred_element_type=jnp.float32) + ib2_ref[...]
    imp = jax.nn.sigmoid(il)
    imp_ref[...] = imp
    factor = 1.0 + (imp > THRESH).astype(jnp.float32)

    csum = jnp.cumsum(sel_f, axis=0)
    pos_f = cnt_ref[...] + csum - sel_f
    cnt_ref[...] = cnt_ref[...] + csum[SB - 1:SB, :]
    pos = pos_f.astype(jnp.int32)

    keep = sel & (pos < CAP)

    cap_iota = jax.lax.broadcasted_iota(jnp.int32, (SB, E, CAP), 2)
    hit = ((cap_iota == pos[:, :, None]) & keep[:, :, None]).astype(jnp.float32)
    disp_ref[...] = hit
    comb_ref[...] = hit * (pnorm * factor)[:, :, None]

    psum_ref[...] = psum_ref[...] + jnp.sum(probs, axis=0, keepdims=True)
    usum_ref[...] = usum_ref[...] + csum[SB - 1:SB, :]

    @pl.when(i == nsteps - 1)
    def _fin():
        prob_mean = psum_ref[...] / S
        usage = usum_ref[...] / (S * K)
        aux_ref[...] = jnp.sum(prob_mean * usage,
                               keepdims=True).reshape(1, 1) * E


def kernel(hidden_states, r_w1, r_b1, r_w2, r_b2,
           imp_w1, imp_b1, imp_w2, imp_b2):
    B = hidden_states.shape[0]
    h2 = hidden_states.reshape(B * S, H)
    grid = (B * S) // SB

    out_shapes = (
        jax.ShapeDtypeStruct((B * S, E, CAP), jnp.float32),
        jax.ShapeDtypeStruct((B * S, E, CAP), jnp.float32),
        jax.ShapeDtypeStruct((B * S, E), jnp.float32),
        jax.ShapeDtypeStruct((B * S, 1), jnp.float32),
        jax.ShapeDtypeStruct((1, 1), jnp.float32),
    )
    full = lambda *shape: pl.BlockSpec(shape, lambda i: (0,) * len(shape))
    outs = pl.pallas_call(
        _router_body,
        grid=(grid,),
        in_specs=[
            pl.BlockSpec((SB, H), lambda i: (i, 0)),
            full(H, H),
            full(1, H),
            full(H, E),
            full(1, E),
            full(H, H // 2),
            full(1, H // 2),
            full(H // 2, 1),
            full(1, 1),
        ],
        out_specs=[
            pl.BlockSpec((SB, E, CAP), lambda i: (i, 0, 0)),
            pl.BlockSpec((SB, E, CAP), lambda i: (i, 0, 0)),
            pl.BlockSpec((SB, E), lambda i: (i, 0)),
            pl.BlockSpec((SB, 1), lambda i: (i, 0)),
            pl.BlockSpec((1, 1), lambda i: (0, 0)),
        ],
        out_shape=out_shapes,
        scratch_shapes=[
            pltpu.VMEM((1, E), jnp.float32),
            pltpu.VMEM((1, E), jnp.float32),
            pltpu.VMEM((1, E), jnp.float32),
        ],
    )(h2, r_w1, r_b1.reshape(1, H), r_w2, r_b2.reshape(1, E),
      imp_w1, imp_b1.reshape(1, H // 2), imp_w2, imp_b2.reshape(1, 1))

    disp, comb, probs, imp, aux = outs
    dispatch = disp.reshape(B, S, E, CAP)
    combine = comb.reshape(B, S, E, CAP)
    router_probs = probs.reshape(B, S, E)
    importance = imp.reshape(B, S)
    aux_loss = aux.reshape(())
    return (dispatch, combine, router_probs, aux_loss, importance)


# speedup vs baseline: 6.2289x; 6.2289x over previous
import jax
import jax.numpy as jnp
from jax.experimental import pallas as pl
from jax.experimental.pallas import tpu as pltpu

S = 2048
H = 1024
E = 8
K = 2
CAP = 768
THRESH = 0.5
SB = 256


def _router_body(h_ref, rw1_ref, rb1_ref, rw2_ref, rb2_ref,
                 iw1_ref, ib1_ref, iw2_ref, ib2_ref,
                 disp_ref, comb_ref, probs_ref, imp_ref, aux_ref,
                 cnt_ref, psum_ref, usum_ref):
    i = pl.program_id(0)
    nsteps = pl.num_programs(0)

    @pl.when(i == 0)
    def _init():
        cnt_ref[...] = jnp.zeros_like(cnt_ref)
        psum_ref[...] = jnp.zeros_like(psum_ref)
        usum_ref[...] = jnp.zeros_like(usum_ref)

    h = h_ref[...]

    rh = jnp.maximum(jnp.dot(h, rw1_ref[...],
                             preferred_element_type=jnp.float32)
                     + rb1_ref[...], 0.0)
    logits = jnp.dot(rh, rw2_ref[...],
                     preferred_element_type=jnp.float32) + rb2_ref[...]

    m = jnp.max(logits, axis=1, keepdims=True)
    ex = jnp.exp(logits - m)
    probs = ex / jnp.sum(ex, axis=1, keepdims=True)
    probs_ref[...] = probs

    eidx = jax.lax.broadcasted_iota(jnp.int32, (SB, E), 1)
    m1 = jnp.max(probs, axis=1, keepdims=True)
    i1 = jnp.min(jnp.where(probs == m1, eidx, E), axis=1, keepdims=True)
    sel1 = eidx == i1
    rest = jnp.where(sel1, -1.0, probs)
    m2 = jnp.max(rest, axis=1, keepdims=True)
    i2 = jnp.min(jnp.where(rest == m2, eidx, E), axis=1, keepdims=True)
    sel2 = eidx == i2
    sel = sel1 | sel2
    sel_f = sel.astype(jnp.float32)

    denom = m1 + m2 + 1e-8
    pnorm = jnp.where(sel1, m1 / denom, 0.0) + jnp.where(sel2, m2 / denom, 0.0)

    ih = jnp.maximum(jnp.dot(h, iw1_ref[...],
                             preferred_element_type=jnp.float32)
                     + ib1_ref[...], 0.0)
    il = jnp.dot(ih, iw2_ref[...],
                 preferred_element_type=jnp.float32) + ib2_ref[...]
    imp = jax.nn.sigmoid(il)
    imp_ref[...] = imp
    factor = 1.0 + (imp > THRESH).astype(jnp.float32)

    r_iota = jax.lax.broadcasted_iota(jnp.int32, (SB, SB), 0)
    c_iota = jax.lax.broadcasted_iota(jnp.int32, (SB, SB), 1)
    tri = (r_iota >= c_iota).astype(jnp.float32)
    csum = jnp.dot(tri, sel_f, preferred_element_type=jnp.float32)
    pos_f = cnt_ref[...] + csum - sel_f
    cnt_ref[...] = cnt_ref[...] + csum[SB - 1:SB, :]
    pos = pos_f.astype(jnp.int32)

    posk = jnp.where(sel & (pos < CAP), pos, -1)

    cap_iota = jax.lax.broadcasted_iota(jnp.int32, (SB, E, CAP), 2)
    hit = (cap_iota == posk[:, :, None]).astype(jnp.float32)
    disp_ref[...] = hit
    comb_ref[...] = hit * (pnorm * factor)[:, :, None]

    psum_ref[...] = psum_ref[...] + jnp.sum(probs, axis=0, keepdims=True)
    usum_ref[...] = usum_ref[...] + csum[SB - 1:SB, :]

    @pl.when(i == nsteps - 1)
    def _fin():
        prob_mean = psum_ref[...] / S
        usage = usum_ref[...] / (S * K)
        aux_ref[...] = jnp.sum(prob_mean * usage,
                               keepdims=True).reshape(1, 1) * E


def kernel(hidden_states, r_w1, r_b1, r_w2, r_b2,
           imp_w1, imp_b1, imp_w2, imp_b2):
    B = hidden_states.shape[0]
    h2 = hidden_states.reshape(B * S, H)
    grid = (B * S) // SB

    out_shapes = (
        jax.ShapeDtypeStruct((B * S, E, CAP), jnp.float32),
        jax.ShapeDtypeStruct((B * S, E, CAP), jnp.float32),
        jax.ShapeDtypeStruct((B * S, E), jnp.float32),
        jax.ShapeDtypeStruct((B * S, 1), jnp.float32),
        jax.ShapeDtypeStruct((1, 1), jnp.float32),
    )
    full = lambda *shape: pl.BlockSpec(shape, lambda i: (0,) * len(shape))
    outs = pl.pallas_call(
        _router_body,
        grid=(grid,),
        in_specs=[
            pl.BlockSpec((SB, H), lambda i: (i, 0)),
            full(H, H),
            full(1, H),
            full(H, E),
            full(1, E),
            full(H, H // 2),
            full(1, H // 2),
            full(H // 2, 1),
            full(1, 1),
        ],
        out_specs=[
            pl.BlockSpec((SB, E, CAP), lambda i: (i, 0, 0)),
            pl.BlockSpec((SB, E, CAP), lambda i: (i, 0, 0)),
            pl.BlockSpec((SB, E), lambda i: (i, 0)),
            pl.BlockSpec((SB, 1), lambda i: (i, 0)),
            pl.BlockSpec((1, 1), lambda i: (0, 0)),
        ],
        out_shape=out_shapes,
        scratch_shapes=[
            pltpu.VMEM((1, E), jnp.float32),
            pltpu.VMEM((1, E), jnp.float32),
            pltpu.VMEM((1, E), jnp.float32),
        ],
    )(h2, r_w1, r_b1.reshape(1, H), r_w2, r_b2.reshape(1, E),
      imp_w1, imp_b1.reshape(1, H // 2), imp_w2, imp_b2.reshape(1, 1))

    disp, comb, probs, imp, aux = outs
    dispatch = disp.reshape(B, S, E, CAP)
    combine = comb.reshape(B, S, E, CAP)
    router_probs = probs.reshape(B, S, E)
    importance = imp.reshape(B, S)
    aux_loss = aux.reshape(())
    return (dispatch, combine, router_probs, aux_loss, importance)
